# baseline (speedup 1.0000x reference)
import functools

import jax
import jax.numpy as jnp
from jax import lax
from jax.experimental import pallas as pl
from jax.experimental.pallas import tpu as pltpu
from jax.experimental.pallas import tpu_sc as plsc

BATCH = 16384
EMBED = 64
PLOT_DIM = 384
HIDDEN = 128
IDX_CHUNK = 128


def _sc_counts():
    try:
        info = plsc.get_sparse_core_info()
        return int(info.num_cores), int(info.num_subcores)
    except Exception:
        return 2, 16


def _make_gather(batch):
    NC, NS = _sc_counts()
    NW = NC * NS
    b_per_w = batch // NW
    n_chunks = b_per_w // IDX_CHUNK
    assert b_per_w % IDX_CHUNK == 0
    rows_per_w = b_per_w // IDX_CHUNK

    mesh = plsc.VectorSubcoreMesh(core_axis_name="c", subcore_axis_name="s")

    @functools.partial(
        pl.kernel,
        out_type=[
            jax.ShapeDtypeStruct((batch, EMBED), jnp.float32),
            jax.ShapeDtypeStruct((batch, EMBED), jnp.float32),
        ],
        mesh=mesh,
        compiler_params=pltpu.CompilerParams(use_tc_tiling_on_sc=False),
        scratch_types=[
            pltpu.VMEM((rows_per_w, IDX_CHUNK), jnp.int32),
            pltpu.VMEM((rows_per_w, IDX_CHUNK), jnp.int32),
            pltpu.VMEM((b_per_w, EMBED), jnp.float32),
            pltpu.VMEM((b_per_w, EMBED), jnp.float32),
            pltpu.SemaphoreType.DMA,
            pltpu.SemaphoreType.DMA,
        ],
    )
    def gather2(users_hbm, movies_hbm, ut_hbm, mt_hbm, ue_out, me_out,
                uidx_v, midx_v, urows_v, mrows_v, usem, msem):
        wid = lax.axis_index("s") * NC + lax.axis_index("c")
        base = wid * b_per_w
        row0 = wid * rows_per_w
        pltpu.sync_copy(users_hbm.at[pl.ds(row0, rows_per_w)], uidx_v)
        pltpu.sync_copy(movies_hbm.at[pl.ds(row0, rows_per_w)], midx_v)
        copies = []
        for j in range(n_chunks):
            dst = urows_v.at[pl.ds(j * IDX_CHUNK, IDX_CHUNK)]
            copies.append(pltpu.async_copy(ut_hbm.at[uidx_v.at[j]], dst, usem))
        for j in range(n_chunks):
            dst = mrows_v.at[pl.ds(j * IDX_CHUNK, IDX_CHUNK)]
            copies.append(pltpu.async_copy(mt_hbm.at[midx_v.at[j]], dst, msem))
        for c in copies:
            c.wait()
        pltpu.sync_copy(urows_v, ue_out.at[pl.ds(base, b_per_w)])
        pltpu.sync_copy(mrows_v, me_out.at[pl.ds(base, b_per_w)])

    return gather2


def _mlp_body(ue_ref, me_ref, plot_ref, w1_ref, b1_ref, w2r_ref, b2_ref,
              out_ref):
    x = jnp.dot(ue_ref[...], w1_ref[0:EMBED, :],
                preferred_element_type=jnp.float32)
    x += jnp.dot(me_ref[...], w1_ref[EMBED:2 * EMBED, :],
                 preferred_element_type=jnp.float32)
    x += jnp.dot(plot_ref[...], w1_ref[2 * EMBED:, :],
                 preferred_element_type=jnp.float32)
    x = jnp.maximum(x + b1_ref[...], 0.0)
    out_ref[...] = (jnp.sum(x * w2r_ref[...], axis=1, keepdims=True)
                    + b2_ref[...])


def _make_mlp(batch, blk):
    grid = batch // blk
    in_dim = 2 * EMBED + PLOT_DIM
    return pl.pallas_call(
        _mlp_body,
        grid=(grid,),
        in_specs=[
            pl.BlockSpec((blk, EMBED), lambda i: (i, 0)),
            pl.BlockSpec((blk, EMBED), lambda i: (i, 0)),
            pl.BlockSpec((blk, PLOT_DIM), lambda i: (i, 0)),
            pl.BlockSpec((in_dim, HIDDEN), lambda i: (0, 0)),
            pl.BlockSpec((1, HIDDEN), lambda i: (0, 0)),
            pl.BlockSpec((1, HIDDEN), lambda i: (0, 0)),
            pl.BlockSpec((1, 1), lambda i: (0, 0)),
        ],
        out_specs=pl.BlockSpec((blk, 1), lambda i: (i, 0)),
        out_shape=jax.ShapeDtypeStruct((batch, 1), jnp.float32),
    )


@jax.jit
def kernel(users, movies, plot_embeddings, user_table, movie_table,
           W1, b1, W2, b2):
    batch = users.shape[0]
    users2d = users.astype(jnp.int32).reshape(-1, IDX_CHUNK)
    movies2d = movies.astype(jnp.int32).reshape(-1, IDX_CHUNK)
    ue, me = _make_gather(batch)(users2d, movies2d, user_table, movie_table)
    mlp = _make_mlp(batch, 2048)
    return mlp(ue, me, plot_embeddings,
               W1, b1.reshape(1, HIDDEN), W2.reshape(1, HIDDEN),
               b2.reshape(1, 1))
